# baseline (speedup 1.0000x reference)
import jax
import jax.numpy as jnp
from jax import lax
from jax.experimental import pallas as pl
from jax.experimental.pallas import tpu as pltpu
from jax.experimental.pallas import tpu_sc as plsc

B = 16384
D = 50
NUM_WORKERS = 32
ROWS_PER_W = B // NUM_WORKERS
C = 64
N_CHUNK = ROWS_PER_W // C


def _gather_body(idx_hbm, tab_hbm, out_hbm, idx_v, rows_v, sem):
    wid = lax.axis_index("s") * 2 + lax.axis_index("c")
    base = wid * ROWS_PER_W
    pltpu.sync_copy(idx_hbm.at[pl.ds(base, ROWS_PER_W)], idx_v)

    def chunk(ci, _):
        cbase = ci * C
        for g in range(C // 16):
            xv = idx_v[pl.ds(cbase + g * 16, 16)]
            for l in range(16):
                k = g * 16 + l
                pltpu.async_copy(
                    tab_hbm.at[pl.ds(xv[l], 1)],
                    rows_v.at[pl.ds(k, 1)],
                    sem)
        pltpu.make_async_copy(tab_hbm.at[pl.ds(0, C)], rows_v, sem).wait()
        pltpu.sync_copy(rows_v, out_hbm.at[pl.ds(base + cbase, C)])
        return 0

    lax.fori_loop(0, N_CHUNK, chunk, 0)


def _sc_gather(idx, table):
    mesh = plsc.VectorSubcoreMesh(core_axis_name="c", subcore_axis_name="s")
    fn = pl.kernel(
        _gather_body,
        out_type=jax.ShapeDtypeStruct((B, D), jnp.float32),
        mesh=mesh,
        scratch_types=[
            pltpu.VMEM((ROWS_PER_W,), jnp.int32),
            pltpu.VMEM((C, D), jnp.float32),
            pltpu.SemaphoreType.DMA,
        ],
        compiler_params=pltpu.CompilerParams(needs_layout_passes=False),
    )
    return fn(idx.astype(jnp.int32), table)


BLK = 2048


def _mlp_body(ue, ve, w1u, w1v, b1, w2, b2, w3, b3, out):
    h1 = jnp.dot(ue[...], w1u[...], preferred_element_type=jnp.float32)
    h1 += jnp.dot(ve[...], w1v[...], preferred_element_type=jnp.float32)
    h1 = jnp.maximum(h1 + b1[...], 0.0)
    h2 = jnp.dot(h1, w2[...], preferred_element_type=jnp.float32)
    h2 = jnp.maximum(h2 + b2[...], 0.0)
    z = jnp.dot(h2, w3[...], preferred_element_type=jnp.float32) + b3[...]
    e = jnp.exp(-jnp.abs(z))
    out[...] = jnp.where(z >= 0.0, 1.0 / (1.0 + e), e / (1.0 + e))


def _mlp(ue, ve, W1, b1, W2, b2, W3, b3):
    w1u = W1[:D]
    w1v = W1[D:]
    grid = B // BLK

    def const(shape):
        return pl.BlockSpec(shape, lambda i: (0, 0))

    return pl.pallas_call(
        _mlp_body,
        grid=(grid,),
        in_specs=[
            pl.BlockSpec((BLK, D), lambda i: (i, 0)),
            pl.BlockSpec((BLK, D), lambda i: (i, 0)),
            const((D, 128)), const((D, 128)), const((1, 128)),
            const((128, 64)), const((1, 64)), const((64, 1)), const((1, 1)),
        ],
        out_specs=pl.BlockSpec((BLK, 1), lambda i: (i, 0)),
        out_shape=jax.ShapeDtypeStruct((B, 1), jnp.float32),
    )(ue, ve, w1u, w1v, b1.reshape(1, 128), W2, b2.reshape(1, 64),
      W3, b3.reshape(1, 1))


def kernel(user_id, video_id, user_table, video_table, W1, b1, W2, b2, W3, b3):
    ve = _sc_gather(video_id, video_table)
    ue = _sc_gather(user_id, user_table)
    return _mlp(ue, ve, W1, b1, W2, b2, W3, b3)


# speedup vs baseline: 1.0034x; 1.0034x over previous
import jax
import jax.numpy as jnp
from jax import lax
from jax.experimental import pallas as pl
from jax.experimental.pallas import tpu as pltpu
from jax.experimental.pallas import tpu_sc as plsc

B = 16384
D = 50
NUM_WORKERS = 32
ROWS_PER_W = B // NUM_WORKERS
C = 64
N_CHUNK = ROWS_PER_W // C


def _gather_body(idx_hbm, tab_hbm, out_hbm, idx_v, rows_v, sem0, sem1):
    wid = lax.axis_index("s") * 2 + lax.axis_index("c")
    base = wid * ROWS_PER_W
    pltpu.sync_copy(idx_hbm.at[pl.ds(base, ROWS_PER_W)], idx_v)
    sems = (sem0, sem1)

    def fire(ci, buf):
        cbase = ci * C
        for g in range(C // 16):
            xv = idx_v[pl.ds(cbase + g * 16, 16)]
            for l in range(16):
                k = g * 16 + l
                pltpu.async_copy(
                    tab_hbm.at[pl.ds(xv[l], 1)],
                    rows_v.at[buf].at[pl.ds(k, 1)],
                    sems[buf])

    def drain(ci, buf):
        pltpu.make_async_copy(tab_hbm.at[pl.ds(0, C)],
                              rows_v.at[buf], sems[buf]).wait()
        pltpu.sync_copy(rows_v.at[buf],
                        out_hbm.at[pl.ds(base + ci * C, C)])

    fire(0, 0)
    for ci in range(N_CHUNK):
        if ci + 1 < N_CHUNK:
            fire(ci + 1, (ci + 1) % 2)
        drain(ci, ci % 2)


def _sc_gather(idx, table):
    mesh = plsc.VectorSubcoreMesh(core_axis_name="c", subcore_axis_name="s")
    fn = pl.kernel(
        _gather_body,
        out_type=jax.ShapeDtypeStruct((B, D), jnp.float32),
        mesh=mesh,
        scratch_types=[
            pltpu.VMEM((ROWS_PER_W,), jnp.int32),
            pltpu.VMEM((2, C, D), jnp.float32),
            pltpu.SemaphoreType.DMA,
            pltpu.SemaphoreType.DMA,
        ],
        compiler_params=pltpu.CompilerParams(needs_layout_passes=False),
    )
    return fn(idx.astype(jnp.int32), table)


BLK = 2048


def _mlp_body(ue, ve, w1u, w1v, b1, w2, b2, w3, b3, out):
    h1 = jnp.dot(ue[...], w1u[...], preferred_element_type=jnp.float32)
    h1 += jnp.dot(ve[...], w1v[...], preferred_element_type=jnp.float32)
    h1 = jnp.maximum(h1 + b1[...], 0.0)
    h2 = jnp.dot(h1, w2[...], preferred_element_type=jnp.float32)
    h2 = jnp.maximum(h2 + b2[...], 0.0)
    z = jnp.dot(h2, w3[...], preferred_element_type=jnp.float32) + b3[...]
    e = jnp.exp(-jnp.abs(z))
    out[...] = jnp.where(z >= 0.0, 1.0 / (1.0 + e), e / (1.0 + e))


def _mlp(ue, ve, W1, b1, W2, b2, W3, b3):
    w1u = W1[:D]
    w1v = W1[D:]
    grid = B // BLK

    def const(shape):
        return pl.BlockSpec(shape, lambda i: (0, 0))

    return pl.pallas_call(
        _mlp_body,
        grid=(grid,),
        in_specs=[
            pl.BlockSpec((BLK, D), lambda i: (i, 0)),
            pl.BlockSpec((BLK, D), lambda i: (i, 0)),
            const((D, 128)), const((D, 128)), const((1, 128)),
            const((128, 64)), const((1, 64)), const((64, 1)), const((1, 1)),
        ],
        out_specs=pl.BlockSpec((BLK, 1), lambda i: (i, 0)),
        out_shape=jax.ShapeDtypeStruct((B, 1), jnp.float32),
    )(ue, ve, w1u, w1v, b1.reshape(1, 128), W2, b2.reshape(1, 64),
      W3, b3.reshape(1, 1))


def kernel(user_id, video_id, user_table, video_table, W1, b1, W2, b2, W3, b3):
    ve = _sc_gather(video_id, video_table)
    ue = _sc_gather(user_id, user_table)
    return _mlp(ue, ve, W1, b1, W2, b2, W3, b3)


# speedup vs baseline: 1.0065x; 1.0031x over previous
import jax
import jax.numpy as jnp
from jax import lax
from jax.experimental import pallas as pl
from jax.experimental.pallas import tpu as pltpu
from jax.experimental.pallas import tpu_sc as plsc

B = 16384
D = 50
NUM_WORKERS = 32
ROWS_PER_W = B // NUM_WORKERS
C = 128
N_CHUNK = ROWS_PER_W // C


def _gather_body(idx_hbm, tab_hbm, out_hbm, idx_v, rows_v, sem0, sem1):
    wid = lax.axis_index("s") * 2 + lax.axis_index("c")
    base = wid * ROWS_PER_W
    pltpu.sync_copy(idx_hbm.at[pl.ds(base, ROWS_PER_W)], idx_v)
    sems = (sem0, sem1)

    def fire(ci, buf):
        cbase = ci * C
        for g in range(C // 16):
            xv = idx_v[pl.ds(cbase + g * 16, 16)]
            for l in range(16):
                k = g * 16 + l
                pltpu.async_copy(
                    tab_hbm.at[pl.ds(xv[l], 1)],
                    rows_v.at[buf].at[pl.ds(k, 1)],
                    sems[buf])

    def drain(ci, buf):
        pltpu.make_async_copy(tab_hbm.at[pl.ds(0, C)],
                              rows_v.at[buf], sems[buf]).wait()
        pltpu.sync_copy(rows_v.at[buf],
                        out_hbm.at[pl.ds(base + ci * C, C)])

    fire(0, 0)
    for ci in range(N_CHUNK):
        if ci + 1 < N_CHUNK:
            fire(ci + 1, (ci + 1) % 2)
        drain(ci, ci % 2)


def _sc_gather(idx, table):
    mesh = plsc.VectorSubcoreMesh(core_axis_name="c", subcore_axis_name="s")
    fn = pl.kernel(
        _gather_body,
        out_type=jax.ShapeDtypeStruct((B, D), jnp.float32),
        mesh=mesh,
        scratch_types=[
            pltpu.VMEM((ROWS_PER_W,), jnp.int32),
            pltpu.VMEM((2, C, D), jnp.float32),
            pltpu.SemaphoreType.DMA,
            pltpu.SemaphoreType.DMA,
        ],
        compiler_params=pltpu.CompilerParams(needs_layout_passes=False),
    )
    return fn(idx.astype(jnp.int32), table)


BLK = 2048


def _mlp_body(ue, ve, w1u, w1v, b1, w2, b2, w3, b3, out):
    h1 = jnp.dot(ue[...], w1u[...], preferred_element_type=jnp.float32)
    h1 += jnp.dot(ve[...], w1v[...], preferred_element_type=jnp.float32)
    h1 = jnp.maximum(h1 + b1[...], 0.0)
    h2 = jnp.dot(h1, w2[...], preferred_element_type=jnp.float32)
    h2 = jnp.maximum(h2 + b2[...], 0.0)
    z = jnp.dot(h2, w3[...], preferred_element_type=jnp.float32) + b3[...]
    e = jnp.exp(-jnp.abs(z))
    out[...] = jnp.where(z >= 0.0, 1.0 / (1.0 + e), e / (1.0 + e))


def _mlp(ue, ve, W1, b1, W2, b2, W3, b3):
    w1u = W1[:D]
    w1v = W1[D:]
    grid = B // BLK

    def const(shape):
        return pl.BlockSpec(shape, lambda i: (0, 0))

    return pl.pallas_call(
        _mlp_body,
        grid=(grid,),
        in_specs=[
            pl.BlockSpec((BLK, D), lambda i: (i, 0)),
            pl.BlockSpec((BLK, D), lambda i: (i, 0)),
            const((D, 128)), const((D, 128)), const((1, 128)),
            const((128, 64)), const((1, 64)), const((64, 1)), const((1, 1)),
        ],
        out_specs=pl.BlockSpec((BLK, 1), lambda i: (i, 0)),
        out_shape=jax.ShapeDtypeStruct((B, 1), jnp.float32),
    )(ue, ve, w1u, w1v, b1.reshape(1, 128), W2, b2.reshape(1, 64),
      W3, b3.reshape(1, 1))


def kernel(user_id, video_id, user_table, video_table, W1, b1, W2, b2, W3, b3):
    ve = _sc_gather(video_id, video_table)
    ue = _sc_gather(user_id, user_table)
    return _mlp(ue, ve, W1, b1, W2, b2, W3, b3)
